# Initial kernel scaffold; baseline (speedup 1.0000x reference)
import jax
import jax.numpy as jnp
from jax.experimental import pallas as pl


def kernel(x_seq, edge_index, edge_weight, W_in, b_in, W1r, b1r, W1u, b1u, W1c, b1c, W2r, b2r, W2u, b2u, W2c, b2c, gamma, beta, W_out, b_out):
    raise NotImplementedError("write your pallas kernel here")


# speedup vs baseline: 5.5146x; 5.5146x over previous
import functools

import jax
import jax.numpy as jnp
from jax import lax
from jax.experimental import pallas as pl
from jax.experimental.pallas import tpu as pltpu
from jax.experimental.pallas import tpu_sc as plsc

_N = 10000
_E = 320000
_FIN = 128
_H = 64
_C = 2
_T = 12

_NSC = 2
_NT = 16
_EPB = 512
_K = _EPB // 128
_NB = 10
_EPAD = _NSC * _NT * _NB * 2 * _EPB
_EPT = _EPAD // (_NSC * _NT)
_NPAD = 10240
_NPS = _NPAD // _NT
_W = 64


def _make_prop():
    mesh = plsc.VectorSubcoreMesh(core_axis_name="c", subcore_axis_name="s")

    @functools.partial(
        pl.kernel,
        out_type=jax.ShapeDtypeStruct((_NSC, _NPAD, _W), jnp.float32),
        mesh=mesh,
        compiler_params=pltpu.CompilerParams(use_tc_tiling_on_sc=False),
        scratch_types=[
            pltpu.VMEM((2 * _K, 128), jnp.int32),
            pltpu.VMEM((2 * _K, 128), jnp.int32),
            pltpu.VMEM((2 * _EPB,), jnp.float32),
            pltpu.VMEM((_EPB, _W), jnp.float32),
            pltpu.VMEM_SHARED((_NPAD, _W), jnp.float32),
            pltpu.SemaphoreType.DMA,
            pltpu.SemaphoreType.DMA,
        ],
    )
    def prop(src_hbm, dst_hbm, ew_hbm, xin_hbm, out_hbm,
             srcv, dstv, ewv, rows, acc, gsem, ssem):
        c = lax.axis_index("c")
        s = lax.axis_index("s")
        tid = c * _NT + s
        zeros16 = jnp.zeros((16,), jnp.float32)

        def zrow(r, carry):
            for k in range(_W // 16):
                rows[r, pl.ds(k * 16, 16)] = zeros16
            return carry

        lax.fori_loop(0, _EPB, zrow, 0)
        row0 = s * _NPS
        done = 0
        while done < _NPS:
            n = min(_EPB, _NPS - done)
            pltpu.sync_copy(rows.at[pl.ds(0, n)], acc.at[pl.ds(row0 + done, n)])
            done += n
        plsc.subcore_barrier()

        def blk(b, carry):
            off = tid * _EPT + b * 2 * _EPB
            r0 = tid * (_EPT // 128) + b * 2 * _K
            pltpu.sync_copy(src_hbm.at[pl.ds(r0, 2 * _K)], srcv)
            pltpu.sync_copy(dst_hbm.at[pl.ds(r0, 2 * _K)], dstv)
            pltpu.sync_copy(ew_hbm.at[pl.ds(off, 2 * _EPB)], ewv)
            for half in range(2):
                gds = [
                    pltpu.async_copy(xin_hbm.at[srcv.at[half * _K + j]],
                                     rows.at[pl.ds(j * 128, 128)], gsem)
                    for j in range(_K)
                ]
                for g in gds:
                    g.wait()

                def scale(g, inner):
                    wv = ewv[pl.ds(half * _EPB + g * 16, 16)]
                    for i in range(16):
                        e = g * 16 + i
                        w = wv[i]
                        for k in range(_W // 16):
                            sl = pl.ds(k * 16, 16)
                            rows[e, sl] = rows[e, sl] * w
                    return inner

                lax.fori_loop(0, _EPB // 16, scale, 0)
                sds = [
                    pltpu.async_copy(rows.at[pl.ds(j * 128, 128)],
                                     acc.at[dstv.at[half * _K + j]], ssem,
                                     add=True)
                    for j in range(_K)
                ]
                for sd in sds:
                    sd.wait()
            return carry

        lax.fori_loop(0, _NB, blk, 0)
        plsc.subcore_barrier()
        pltpu.sync_copy(acc.at[pl.ds(row0, _NPS)],
                        out_hbm.at[c].at[pl.ds(row0, _NPS)])

    return prop


_BN = 1000
_G = _N // _BN


def _dinv_of(dg_ref):
    return lax.rsqrt(dg_ref[0, :, :1] + dg_ref[1, :, :1] + 1.0)


def _embed_body(x_ref, w_ref, b_ref, dg_ref, o_ref):
    dinv = _dinv_of(dg_ref)
    w = w_ref[...]
    bias = b_ref[...]
    for t in range(_T):
        y = jnp.dot(x_ref[t], w, preferred_element_type=jnp.float32) + bias
        o_ref[:, t * _H:(t + 1) * _H] = dinv * jnp.maximum(y, 0.0)


def _embed(x_seq, W_in, b_in, degp):
    return pl.pallas_call(
        _embed_body,
        grid=(_G,),
        in_specs=[
            pl.BlockSpec((_T, _BN, _FIN), lambda b: (0, b, 0)),
            pl.BlockSpec((_FIN, _H), lambda b: (0, 0)),
            pl.BlockSpec((1, _H), lambda b: (0, 0)),
            pl.BlockSpec((_NSC, _BN, 8), lambda b: (0, b, 0)),
        ],
        out_specs=pl.BlockSpec((_BN, _T * _H), lambda b: (b, 0)),
        out_shape=jax.ShapeDtypeStruct((_N, _T * _H), jnp.float32),
    )(x_seq, W_in, b_in.reshape(1, _H), degp)


def _gates_body(zx_ref, xt_ref, zh_ref, hs_ref, dg_ref, wr_ref, br_ref,
                wu_ref, bu_ref, rh_ref, u_ref, px_ref):
    dinv = _dinv_of(dg_ref)
    px = dinv * (zx_ref[0] + zx_ref[1] + xt_ref[...])
    ph = dinv * (zh_ref[0] + zh_ref[1] + hs_ref[...])
    wr = wr_ref[...]
    wu = wu_ref[...]
    pre_r = (jnp.dot(px, wr[:_H], preferred_element_type=jnp.float32)
             + jnp.dot(ph, wr[_H:], preferred_element_type=jnp.float32)
             + br_ref[...])
    pre_u = (jnp.dot(px, wu[:_H], preferred_element_type=jnp.float32)
             + jnp.dot(ph, wu[_H:], preferred_element_type=jnp.float32)
             + bu_ref[...])
    r = jax.nn.sigmoid(pre_r)
    rh_ref[...] = r * hs_ref[...]
    u_ref[...] = jax.nn.sigmoid(pre_u)
    px_ref[...] = px


def _gates(Zx, xt, Zh, hs, degp, Wr, br, Wu, bu):
    vec = pl.BlockSpec((_BN, _H), lambda b: (b, 0))
    par = pl.BlockSpec((_NSC, _BN, _H), lambda b: (0, b, 0))
    return pl.pallas_call(
        _gates_body,
        grid=(_G,),
        in_specs=[
            par, vec, par, vec,
            pl.BlockSpec((_NSC, _BN, 8), lambda b: (0, b, 0)),
            pl.BlockSpec((2 * _H, _H), lambda b: (0, 0)),
            pl.BlockSpec((1, _H), lambda b: (0, 0)),
            pl.BlockSpec((2 * _H, _H), lambda b: (0, 0)),
            pl.BlockSpec((1, _H), lambda b: (0, 0)),
        ],
        out_specs=[vec, vec, vec],
        out_shape=[jax.ShapeDtypeStruct((_N, _H), jnp.float32)] * 3,
    )(Zx, xt, Zh, hs, degp, Wr, br.reshape(1, _H), Wu, bu.reshape(1, _H))


def _update_body(zc_ref, rh_ref, px_ref, u_ref, h_ref, dg_ref, wc_ref, bc_ref,
                 hn_ref, hs_ref):
    dinv = _dinv_of(dg_ref)
    pc = dinv * (zc_ref[0] + zc_ref[1] + rh_ref[...])
    wc = wc_ref[...]
    pre_c = (jnp.dot(px_ref[...], wc[:_H], preferred_element_type=jnp.float32)
             + jnp.dot(pc, wc[_H:], preferred_element_type=jnp.float32)
             + bc_ref[...])
    cg = jnp.tanh(pre_c)
    u = u_ref[...]
    hn = u * h_ref[...] + (1.0 - u) * cg
    hn_ref[...] = hn
    hs_ref[...] = dinv * hn


def _update(Zc, rh, px, u, h, degp, Wc, bc):
    vec = pl.BlockSpec((_BN, _H), lambda b: (b, 0))
    par = pl.BlockSpec((_NSC, _BN, _H), lambda b: (0, b, 0))
    return pl.pallas_call(
        _update_body,
        grid=(_G,),
        in_specs=[
            par, vec, vec, vec, vec,
            pl.BlockSpec((_NSC, _BN, 8), lambda b: (0, b, 0)),
            pl.BlockSpec((2 * _H, _H), lambda b: (0, 0)),
            pl.BlockSpec((1, _H), lambda b: (0, 0)),
        ],
        out_specs=[vec, vec],
        out_shape=[jax.ShapeDtypeStruct((_N, _H), jnp.float32)] * 2,
    )(Zc, rh, px, u, h, degp, Wc, bc.reshape(1, _H))


def _stats_body(h_ref, o_ref):
    @pl.when(pl.program_id(0) == 0)
    def _():
        o_ref[...] = jnp.zeros_like(o_ref)

    hb = h_ref[...]
    s = jnp.sum(hb, axis=0)[None, :]
    s2 = jnp.sum(hb * hb, axis=0)[None, :]
    o_ref[0:2, :] += jnp.concatenate([s, s2], axis=0)


def _stats(h):
    return pl.pallas_call(
        _stats_body,
        grid=(_G,),
        in_specs=[pl.BlockSpec((_BN, _H), lambda b: (b, 0))],
        out_specs=pl.BlockSpec((8, _H), lambda b: (0, 0)),
        out_shape=jax.ShapeDtypeStruct((8, _H), jnp.float32),
    )(h)


def _head_body(h_ref, st_ref, g_ref, be_ref, wo_ref, bo_ref, o_ref):
    mean = st_ref[0:1, :] / _N
    ex2 = st_ref[1:2, :] / _N
    var = ex2 - mean * mean
    scale = g_ref[...] * lax.rsqrt(var + 1e-5)
    hn = jnp.maximum((h_ref[...] - mean) * scale + be_ref[...], 0.0)
    y = jnp.dot(hn, wo_ref[...], preferred_element_type=jnp.float32) + bo_ref[...]
    m = jnp.max(y, axis=1, keepdims=True)
    z = y - m
    lse = jnp.log(jnp.sum(jnp.exp(z), axis=1, keepdims=True))
    o_ref[...] = z - lse


def _head(h, stats, gamma, beta, W_out, b_out):
    return pl.pallas_call(
        _head_body,
        grid=(_G,),
        in_specs=[
            pl.BlockSpec((_BN, _H), lambda b: (b, 0)),
            pl.BlockSpec((8, _H), lambda b: (0, 0)),
            pl.BlockSpec((1, _H), lambda b: (0, 0)),
            pl.BlockSpec((1, _H), lambda b: (0, 0)),
            pl.BlockSpec((_H, _C), lambda b: (0, 0)),
            pl.BlockSpec((1, _C), lambda b: (0, 0)),
        ],
        out_specs=pl.BlockSpec((_BN, _C), lambda b: (b, 0)),
        out_shape=jax.ShapeDtypeStruct((_N, _C), jnp.float32),
    )(h, stats, gamma.reshape(1, _H), beta.reshape(1, _H), W_out,
      b_out.reshape(1, _C))


def kernel(x_seq, edge_index, edge_weight, W_in, b_in, W1r, b1r, W1u, b1u,
           W1c, b1c, W2r, b2r, W2u, b2u, W2c, b2c, gamma, beta, W_out, b_out):
    pad = _EPAD - _E
    src_p = jnp.concatenate(
        [edge_index[0], jnp.zeros((pad,), jnp.int32)]).reshape(_EPAD // 128, 128)
    dst_p = jnp.concatenate(
        [edge_index[1], jnp.zeros((pad,), jnp.int32)]).reshape(_EPAD // 128, 128)
    ew_p = jnp.concatenate([edge_weight, jnp.zeros((pad,), jnp.float32)])

    propk = _make_prop()

    def prop(a):
        return propk(src_p, dst_p, ew_p, a)

    degp = prop(jnp.ones((_N, _W), jnp.float32))[:, :, :8]
    Xs = _embed(x_seq, W_in, b_in, degp)

    Zx_all = [prop(Xs[:, i * _H:(i + 1) * _H]) for i in range(_T)]

    z64 = jnp.zeros((_NSC, _NPAD, _H), jnp.float32)
    zN = jnp.zeros((_N, _H), jnp.float32)

    rh1, u1, px1 = _gates(Zx_all[0], Xs[:, :_H], z64, zN, degp,
                          W1r, b1r, W1u, b1u)
    h1, h1s = _update(z64, rh1, px1, u1, zN, degp, W1c, b1c)
    P1 = prop(h1s)
    rh2, u2, px2 = _gates(P1, h1s, z64, zN, degp, W2r, b2r, W2u, b2u)
    h2, h2s = _update(z64, rh2, px2, u2, zN, degp, W2c, b2c)

    for t in range(1, _T):
        Zh2 = prop(h2s)
        rh1, u1, px1 = _gates(Zx_all[t], Xs[:, t * _H:(t + 1) * _H], P1,
                              h1s, degp, W1r, b1r, W1u, b1u)
        h1, h1s = _update(prop(rh1), rh1, px1, u1, h1, degp, W1c, b1c)
        P1 = prop(h1s)
        rh2, u2, px2 = _gates(P1, h1s, Zh2, h2s, degp, W2r, b2r, W2u, b2u)
        h2, h2s = _update(prop(rh2), rh2, px2, u2, h2, degp, W2c, b2c)

    st = _stats(h2)
    return _head(h2, st, gamma, beta, W_out, b_out)
